# Initial kernel scaffold; baseline (speedup 1.0000x reference)
import jax
import jax.numpy as jnp
from jax.experimental import pallas as pl


def kernel(input_ids, inputs_embeds, vision_embeddings, image_token_id):
    raise NotImplementedError("write your pallas kernel here")


# speedup vs baseline: 2.3984x; 2.3984x over previous
import jax
import jax.numpy as jnp
from jax.experimental import pallas as pl
from jax.experimental.pallas import tpu as pltpu

_R = 512


def _body(tok_ref, ids_ref, emb_hbm, vis_hbm, out_ref,
          vbuf, ebuf, vsem, esem, carry):
    j = pl.program_id(0)
    nb = pl.num_programs(0)
    R = ebuf.shape[1]
    n = vis_hbm.shape[0]
    tok = tok_ref[0]

    def vis_start(off):
        aligned = (off // 8) * 8
        return pl.multiple_of(jnp.minimum(aligned, n - (R + 8)), 8)

    def block_count(i):
        m = (ids_ref[pl.ds(i, 1), :] == tok)
        return jnp.sum(m.astype(jnp.int32))

    def start_dmas(slot, off, k, base):
        @pl.when(k > 0)
        def _():
            pltpu.make_async_copy(
                vis_hbm.at[pl.ds(vis_start(off), R + 8), :],
                vbuf.at[slot], vsem.at[slot]
            ).start()

        @pl.when(k < R)
        def _():
            pltpu.make_async_copy(
                emb_hbm.at[pl.ds(base, R), :], ebuf.at[slot], esem.at[slot]
            ).start()

    @pl.when(j == 0)
    def _():
        carry[0] = 0

    off = carry[0]
    k = block_count(j)

    @pl.when(j == 0)
    def _():
        start_dmas(0, 0, k, 0)

    @pl.when(j + 1 < nb)
    def _():
        kn = block_count(j + 1)
        start_dmas((j + 1) % 2, off + k, kn, (j + 1) * R)

    carry[0] = off + k

    slot = j % 2
    shift = off - vis_start(off)

    def vis_rows():
        nrows = R + 8
        rolled = pltpu.roll(vbuf[slot], (nrows - shift) % nrows, 0)
        return rolled[:R, :]

    @pl.when(k > 0)
    def _():
        pltpu.make_async_copy(
            vis_hbm.at[pl.ds(vis_start(off), R + 8), :],
            vbuf.at[slot], vsem.at[slot]
        ).wait()

    @pl.when(k < R)
    def _():
        pltpu.make_async_copy(
            emb_hbm.at[pl.ds(j * R, R), :], ebuf.at[slot], esem.at[slot]
        ).wait()

    @pl.when(jnp.logical_and(k == R, shift == 0))
    def _():
        out_ref[...] = vbuf[slot, :R, :]

    @pl.when(jnp.logical_and(k == R, shift != 0))
    def _():
        out_ref[...] = vis_rows()

    @pl.when(k == 0)
    def _():
        out_ref[...] = ebuf[slot]

    @pl.when(jnp.logical_and(k > 0, k < R))
    def _():
        m_row = (ids_ref[pl.ds(j, 1), :] == tok)
        m_f = m_row.astype(jnp.float32)
        upper = (jax.lax.broadcasted_iota(jnp.int32, (R, R), 0)
                 < jax.lax.broadcasted_iota(jnp.int32, (R, R), 1))
        ranks = jnp.dot(m_f, upper.astype(jnp.float32),
                        preferred_element_type=jnp.float32)
        qidx = jax.lax.broadcasted_iota(jnp.int32, (R, R), 0)
        Pt = jnp.where(jnp.logical_and(m_row, ranks.astype(jnp.int32) == qidx),
                       jnp.float32(1.0), jnp.float32(0.0))
        g = jax.lax.dot_general(
            Pt, vis_rows(),
            dimension_numbers=(((0,), (0,)), ((), ())),
            preferred_element_type=jnp.float32)
        ones_col = jnp.ones((R, 1), jnp.float32)
        keep = jnp.float32(1.0) - jax.lax.dot_general(
            Pt, ones_col,
            dimension_numbers=(((0,), (0,)), ((), ())),
            preferred_element_type=jnp.float32)
        out_ref[...] = g + ebuf[slot] * keep


def kernel(input_ids, inputs_embeds, vision_embeddings, image_token_id):
    b, s, d = inputs_embeds.shape
    n = b * s
    nb = n // _R
    ids = input_ids.reshape(nb, _R)
    tok = jnp.asarray(image_token_id, jnp.int32).reshape(1)
    embeds = inputs_embeds.reshape(n, d)
    vis = vision_embeddings.reshape(
        vision_embeddings.shape[0] * vision_embeddings.shape[1], d)

    out = pl.pallas_call(
        _body,
        grid=(nb,),
        in_specs=[
            pl.BlockSpec(memory_space=pltpu.SMEM),
            pl.BlockSpec(memory_space=pltpu.VMEM),
            pl.BlockSpec(memory_space=pl.ANY),
            pl.BlockSpec(memory_space=pl.ANY),
        ],
        out_specs=pl.BlockSpec((_R, d), lambda j: (j, 0)),
        out_shape=jax.ShapeDtypeStruct((n, d), jnp.float32),
        scratch_shapes=[
            pltpu.VMEM((2, _R + 8, d), jnp.float32),
            pltpu.VMEM((2, _R, d), jnp.float32),
            pltpu.SemaphoreType.DMA((2,)),
            pltpu.SemaphoreType.DMA((2,)),
            pltpu.SMEM((1,), jnp.int32),
        ],
    )(tok, ids, embeds, vis)
    return out.reshape(b, s, d)


# speedup vs baseline: 2.6493x; 1.1046x over previous
import jax
import jax.numpy as jnp
from jax.experimental import pallas as pl
from jax.experimental.pallas import tpu as pltpu

_R = 1024


def _body(tok_ref, ids_ref, emb_hbm, vis_hbm, out_ref,
          vbuf, ebuf, vsem, esem, carry):
    j = pl.program_id(0)
    nb = pl.num_programs(0)
    R = ebuf.shape[1]
    n = vis_hbm.shape[0]
    tok = tok_ref[0]

    def vis_start(off):
        aligned = (off // 8) * 8
        return pl.multiple_of(jnp.minimum(aligned, n - (R + 8)), 8)

    def block_count(i):
        m = (ids_ref[pl.ds(i, 1), :] == tok)
        return jnp.sum(m.astype(jnp.int32))

    def start_dmas(slot, off, k, base):
        @pl.when(k > 0)
        def _():
            pltpu.make_async_copy(
                vis_hbm.at[pl.ds(vis_start(off), R + 8), :],
                vbuf.at[slot], vsem.at[slot]
            ).start()

        @pl.when(k < R)
        def _():
            pltpu.make_async_copy(
                emb_hbm.at[pl.ds(base, R), :], ebuf.at[slot], esem.at[slot]
            ).start()

    @pl.when(j == 0)
    def _():
        carry[0] = 0

    off = carry[0]
    k = block_count(j)

    @pl.when(j == 0)
    def _():
        start_dmas(0, 0, k, 0)

    @pl.when(j + 1 < nb)
    def _():
        kn = block_count(j + 1)
        start_dmas((j + 1) % 2, off + k, kn, (j + 1) * R)

    carry[0] = off + k

    slot = j % 2
    shift = off - vis_start(off)

    def vis_rows():
        nrows = R + 8
        rolled = pltpu.roll(vbuf[slot], (nrows - shift) % nrows, 0)
        return rolled[:R, :]

    @pl.when(k > 0)
    def _():
        pltpu.make_async_copy(
            vis_hbm.at[pl.ds(vis_start(off), R + 8), :],
            vbuf.at[slot], vsem.at[slot]
        ).wait()

    @pl.when(k < R)
    def _():
        pltpu.make_async_copy(
            emb_hbm.at[pl.ds(j * R, R), :], ebuf.at[slot], esem.at[slot]
        ).wait()

    @pl.when(jnp.logical_and(k == R, shift == 0))
    def _():
        out_ref[...] = vbuf[slot, :R, :]

    @pl.when(jnp.logical_and(k == R, shift != 0))
    def _():
        out_ref[...] = vis_rows()

    @pl.when(k == 0)
    def _():
        out_ref[...] = ebuf[slot]

    @pl.when(jnp.logical_and(k > 0, k < R))
    def _():
        m_row = (ids_ref[pl.ds(j, 1), :] == tok)
        m_f = m_row.astype(jnp.float32)
        upper = (jax.lax.broadcasted_iota(jnp.int32, (R, R), 0)
                 < jax.lax.broadcasted_iota(jnp.int32, (R, R), 1))
        ranks = jnp.dot(m_f, upper.astype(jnp.float32),
                        preferred_element_type=jnp.float32)
        qidx = jax.lax.broadcasted_iota(jnp.int32, (R, R), 0)
        Pt = jnp.where(jnp.logical_and(m_row, ranks.astype(jnp.int32) == qidx),
                       jnp.float32(1.0), jnp.float32(0.0))
        g = jax.lax.dot_general(
            Pt, vis_rows(),
            dimension_numbers=(((0,), (0,)), ((), ())),
            preferred_element_type=jnp.float32)
        ones_col = jnp.ones((R, 1), jnp.float32)
        keep = jnp.float32(1.0) - jax.lax.dot_general(
            Pt, ones_col,
            dimension_numbers=(((0,), (0,)), ((), ())),
            preferred_element_type=jnp.float32)
        out_ref[...] = g + ebuf[slot] * keep


def kernel(input_ids, inputs_embeds, vision_embeddings, image_token_id):
    b, s, d = inputs_embeds.shape
    n = b * s
    nb = n // _R
    ids = input_ids.reshape(nb, _R)
    tok = jnp.asarray(image_token_id, jnp.int32).reshape(1)
    embeds = inputs_embeds.reshape(n, d)
    vis = vision_embeddings.reshape(
        vision_embeddings.shape[0] * vision_embeddings.shape[1], d)

    out = pl.pallas_call(
        _body,
        grid=(nb,),
        in_specs=[
            pl.BlockSpec(memory_space=pltpu.SMEM),
            pl.BlockSpec(memory_space=pltpu.VMEM),
            pl.BlockSpec(memory_space=pl.ANY),
            pl.BlockSpec(memory_space=pl.ANY),
        ],
        out_specs=pl.BlockSpec((_R, d), lambda j: (j, 0)),
        out_shape=jax.ShapeDtypeStruct((n, d), jnp.float32),
        scratch_shapes=[
            pltpu.VMEM((2, _R + 8, d), jnp.float32),
            pltpu.VMEM((2, _R, d), jnp.float32),
            pltpu.SemaphoreType.DMA((2,)),
            pltpu.SemaphoreType.DMA((2,)),
            pltpu.SMEM((1,), jnp.int32),
        ],
    )(tok, ids, embeds, vis)
    return out.reshape(b, s, d)
